# baseline (speedup 1.0000x reference)
import dataclasses
import functools

import jax
import jax.numpy as jnp
from jax import lax
from jax.experimental import pallas as pl
from jax.experimental.pallas import tpu as pltpu
from jax.experimental.pallas import tpu_sc as plsc

N_NODES = 10000
N_EDGES = 320000
D_IN = 128
D_HID = 128
N_CLASSES = 64

NCORE = 2
NSUB = 16
NW = NCORE * NSUB
K = 96
NBUF = 3
NCH = 105
E_PAD = NW * NCH * K
N_PAD = 10240
ROWS_PER_SUB = N_PAD // NSUB


def _linear1(x, W1, b1):
    def body(x_ref, w_ref, b_ref, o_ref):
        o_ref[...] = lax.dot_general(
            x_ref[...], w_ref[...], (((1,), (1,)), ((), ())),
            preferred_element_type=jnp.float32) + b_ref[...]

    return pl.pallas_call(
        body,
        grid=(10,),
        in_specs=[
            pl.BlockSpec((N_NODES // 10, D_IN), lambda i: (i, 0)),
            pl.BlockSpec((D_HID, D_IN), lambda i: (0, 0)),
            pl.BlockSpec((1, D_HID), lambda i: (0, 0)),
        ],
        out_specs=pl.BlockSpec((N_NODES // 10, D_HID), lambda i: (i, 0)),
        out_shape=jax.ShapeDtypeStruct((N_NODES, D_HID), jnp.float32),
    )(x, W1, b1)


def _relu_linear2(p, W2p, b2p):
    def body(p_ref, w_ref, b_ref, o_ref):
        h = jnp.maximum(p_ref[0] + p_ref[1], 0.0)
        o_ref[...] = lax.dot_general(
            h, w_ref[...], (((1,), (1,)), ((), ())),
            preferred_element_type=jnp.float32) + b_ref[...]

    return pl.pallas_call(
        body,
        grid=(10,),
        in_specs=[
            pl.BlockSpec((NCORE, N_NODES // 10, D_HID), lambda i: (0, i, 0)),
            pl.BlockSpec((D_HID, D_HID), lambda i: (0, 0)),
            pl.BlockSpec((1, D_HID), lambda i: (0, 0)),
        ],
        out_specs=pl.BlockSpec((N_NODES // 10, D_HID), lambda i: (i, 0)),
        out_shape=jax.ShapeDtypeStruct((N_NODES, D_HID), jnp.float32),
    )(p, W2p, b2p)


def _sum_partials(q):
    def body(q_ref, o_ref):
        o_ref[...] = q_ref[0, :, :N_CLASSES] + q_ref[1, :, :N_CLASSES]

    return pl.pallas_call(
        body,
        grid=(10,),
        in_specs=[pl.BlockSpec((NCORE, N_NODES // 10, D_HID),
                               lambda i: (0, i, 0))],
        out_specs=pl.BlockSpec((N_NODES // 10, N_CLASSES), lambda i: (i, 0)),
        out_shape=jax.ShapeDtypeStruct((N_NODES, N_CLASSES), jnp.float32),
    )(q)


def _spmm_sc(h, edges, zeros, d, d_active):
    mesh = plsc.VectorSubcoreMesh(core_axis_name="c", subcore_axis_name="s")
    cp = pltpu.CompilerParams()
    if "needs_layout_passes" in pltpu.CompilerParams.__dataclass_fields__:
        cp = dataclasses.replace(cp, needs_layout_passes=False)

    @functools.partial(
        pl.kernel,
        out_type=jax.ShapeDtypeStruct((NCORE, N_PAD, d), jnp.float32),
        mesh=mesh,
        compiler_params=cp,
        scratch_types=[
            pltpu.VMEM_SHARED((N_PAD, d), jnp.float32),
        ] + [pltpu.VMEM((K, d), jnp.float32) for _ in range(NBUF)]
          + [pltpu.VMEM((3, K), jnp.int32) for _ in range(NBUF)]
          + [pltpu.VMEM((1, K), jnp.int32) for _ in range(NBUF)]
          + [pltpu.SemaphoreType.DMA for _ in range(3 * NBUF)],
    )
    def k(h_hbm, e_hbm, z_hbm, out_hbm, acc_sh, *bufs_and_sems):
        rows = bufs_and_sems[:NBUF]
        ib = bufs_and_sems[NBUF:2 * NBUF]
        dstb = bufs_and_sems[2 * NBUF:3 * NBUF]
        sems = bufs_and_sems[3 * NBUF:]
        gsem = sems[:NBUF]
        ssem = sems[NBUF:2 * NBUF]
        isem = sems[2 * NBUF:]
        cid = lax.axis_index("c")
        sid = lax.axis_index("s")
        wid = cid * NSUB + sid

        pltpu.sync_copy(z_hbm.at[pl.ds(sid * ROWS_PER_SUB, ROWS_PER_SUB)],
                        acc_sh.at[pl.ds(sid * ROWS_PER_SUB, ROWS_PER_SUB)])
        plsc.subcore_barrier()

        def idx_start(c, b):
            pltpu.async_copy(e_hbm.at[wid].at[c], ib[b], isem[b])

        def idx_wait(c, b):
            pltpu.make_async_copy(e_hbm.at[wid].at[c], ib[b], isem[b]).wait()

        def gather_start(c, b):
            pltpu.async_copy(h_hbm.at[ib[b].at[0]], rows[b], gsem[b])

        def gather_wait(c, b):
            pltpu.make_async_copy(h_hbm.at[ib[b].at[0]], rows[b],
                                  gsem[b]).wait()

        def scatter_start(c, b):
            pltpu.async_copy(rows[b], acc_sh.at[dstb[b].at[0]], ssem[b],
                             add=True)

        def scatter_wait(c, b):
            pltpu.make_async_copy(rows[b], acc_sh.at[dstb[b].at[0]],
                                  ssem[b]).wait()

        def scale(b):
            two = jnp.full((16,), 2, jnp.int32)

            @plsc.parallel_loop(0, K, step=8, unroll=2)
            def _(e0):
                base = jnp.full((16,), e0, jnp.int32)
                for t in range(8):
                    ee = base + t
                    wsplat = plsc.bitcast(
                        plsc.load_gather(ib[b], [two, ee]), jnp.float32)
                    for g in range(d_active // 16):
                        sl = (e0 + t, pl.ds(g * 16, 16))
                        rows[b][sl] = rows[b][sl] * wsplat

        def save_dst(b):
            for g in range(K // 16):
                dstb[b][0, pl.ds(g * 16, 16)] = ib[b][1, pl.ds(g * 16, 16)]

        idx_start(0, 0)
        idx_start(1, 1)
        idx_start(2, 2)
        idx_wait(0, 0)
        gather_start(0, 0)
        idx_wait(1, 1)
        gather_start(1, 1)

        @pl.loop(0, NCH, step=NBUF)
        def _(c0):
            for b in range(NBUF):
                c = c0 + b
                gather_wait(c, b)
                scale(b)
                save_dst(b)
                scatter_start(c, b)

                @pl.when(c + 3 < NCH)
                def _():
                    idx_start(c + 3, b)

                cn = c + 2

                @pl.when(cn < NCH)
                def _():
                    @pl.when(c >= 1)
                    def _():
                        scatter_wait(c - 1, (b - 1) % NBUF)
                    idx_wait(cn, (b + 2) % NBUF)
                    gather_start(cn, (b + 2) % NBUF)

        for i in range(NBUF):
            c = NCH - NBUF + i
            scatter_wait(c, c % NBUF)

        plsc.subcore_barrier()

        pltpu.sync_copy(
            acc_sh.at[pl.ds(sid * ROWS_PER_SUB, ROWS_PER_SUB)],
            out_hbm.at[cid].at[pl.ds(sid * ROWS_PER_SUB, ROWS_PER_SUB)])

    return k(h, edges, zeros)


def kernel(x, edge_index, edge_weight, W1, b1, W2, b2):
    pad = E_PAD - N_EDGES
    pad_dst = N_NODES + (jnp.arange(pad, dtype=jnp.int32) % (N_PAD - N_NODES))
    pad_src = jnp.arange(pad, dtype=jnp.int32) % N_NODES
    dst = jnp.concatenate(
        [edge_index[0].astype(jnp.int32), pad_dst]).reshape(NW, NCH, K)
    src = jnp.concatenate(
        [edge_index[1].astype(jnp.int32), pad_src]).reshape(NW, NCH, K)
    wbits = lax.bitcast_convert_type(
        jnp.pad(edge_weight, (0, pad)), jnp.int32).reshape(NW, NCH, K)
    edges = jnp.stack([src, dst, wbits], axis=2)
    z1 = jnp.zeros((N_PAD, D_HID), jnp.float32)
    W2p = jnp.pad(W2, ((0, D_HID - N_CLASSES), (0, 0)))
    b2p = jnp.pad(b2, (0, D_HID - N_CLASSES)).reshape(1, D_HID)

    h = _linear1(x, W1, b1.reshape(1, D_HID))
    p = _spmm_sc(h, edges, z1, D_HID, D_HID)
    h2 = _relu_linear2(p, W2p, b2p)
    q = _spmm_sc(h2, edges, z1, D_HID, N_CLASSES)
    return _sum_partials(q)


# speedup vs baseline: 1.0076x; 1.0076x over previous
import dataclasses
import functools

import jax
import jax.numpy as jnp
from jax import lax
from jax.experimental import pallas as pl
from jax.experimental.pallas import tpu as pltpu
from jax.experimental.pallas import tpu_sc as plsc

N_NODES = 10000
N_EDGES = 320000
D_IN = 128
D_HID = 128
N_CLASSES = 64

NCORE = 2
NSUB = 16
NW = NCORE * NSUB
K = 96
NBUF = 3
NCH = 105
E_PAD = NW * NCH * K
N_PAD = 10240
ROWS_PER_SUB = N_PAD // NSUB


def _linear1(x, W1, b1):
    def body(x_ref, w_ref, b_ref, o_ref):
        o_ref[...] = lax.dot_general(
            x_ref[...], w_ref[...], (((1,), (1,)), ((), ())),
            preferred_element_type=jnp.float32) + b_ref[...]

    return pl.pallas_call(
        body,
        grid=(10,),
        in_specs=[
            pl.BlockSpec((N_NODES // 10, D_IN), lambda i: (i, 0)),
            pl.BlockSpec((D_HID, D_IN), lambda i: (0, 0)),
            pl.BlockSpec((1, D_HID), lambda i: (0, 0)),
        ],
        out_specs=pl.BlockSpec((N_NODES // 10, D_HID), lambda i: (i, 0)),
        out_shape=jax.ShapeDtypeStruct((N_NODES, D_HID), jnp.float32),
    )(x, W1, b1)


def _relu_linear2(p, W2p, b2p):
    def body(p_ref, w_ref, b_ref, o_ref):
        h = jnp.maximum(p_ref[0] + p_ref[1], 0.0)
        o_ref[...] = lax.dot_general(
            h, w_ref[...], (((1,), (1,)), ((), ())),
            preferred_element_type=jnp.float32) + b_ref[...]

    return pl.pallas_call(
        body,
        grid=(10,),
        in_specs=[
            pl.BlockSpec((NCORE, N_NODES // 10, D_HID), lambda i: (0, i, 0)),
            pl.BlockSpec((D_HID, D_HID), lambda i: (0, 0)),
            pl.BlockSpec((1, D_HID), lambda i: (0, 0)),
        ],
        out_specs=pl.BlockSpec((N_NODES // 10, D_HID), lambda i: (i, 0)),
        out_shape=jax.ShapeDtypeStruct((N_NODES, D_HID), jnp.float32),
    )(p, W2p, b2p)


def _sum_partials(q):
    def body(q_ref, o_ref):
        o_ref[...] = q_ref[0, :, :N_CLASSES] + q_ref[1, :, :N_CLASSES]

    return pl.pallas_call(
        body,
        grid=(10,),
        in_specs=[pl.BlockSpec((NCORE, N_NODES // 10, D_HID),
                               lambda i: (0, i, 0))],
        out_specs=pl.BlockSpec((N_NODES // 10, N_CLASSES), lambda i: (i, 0)),
        out_shape=jax.ShapeDtypeStruct((N_NODES, N_CLASSES), jnp.float32),
    )(q)


def _spmm_sc(h, edges, zeros, d, d_active):
    mesh = plsc.VectorSubcoreMesh(core_axis_name="c", subcore_axis_name="s")
    cp = pltpu.CompilerParams()
    if "needs_layout_passes" in pltpu.CompilerParams.__dataclass_fields__:
        cp = dataclasses.replace(cp, needs_layout_passes=False)

    @functools.partial(
        pl.kernel,
        out_type=jax.ShapeDtypeStruct((NCORE, N_PAD, d), jnp.float32),
        mesh=mesh,
        compiler_params=cp,
        scratch_types=[
            pltpu.VMEM_SHARED((N_PAD, d), jnp.float32),
        ] + [pltpu.VMEM((K, d), jnp.float32) for _ in range(NBUF)]
          + [pltpu.VMEM((3, K), jnp.int32) for _ in range(NBUF)]
          + [pltpu.VMEM((1, K), jnp.int32) for _ in range(NBUF)]
          + [pltpu.SemaphoreType.DMA for _ in range(3 * NBUF)],
    )
    def k(h_hbm, e_hbm, z_hbm, out_hbm, acc_sh, *bufs_and_sems):
        rows = bufs_and_sems[:NBUF]
        ib = bufs_and_sems[NBUF:2 * NBUF]
        dstb = bufs_and_sems[2 * NBUF:3 * NBUF]
        sems = bufs_and_sems[3 * NBUF:]
        gsem = sems[:NBUF]
        ssem = sems[NBUF:2 * NBUF]
        isem = sems[2 * NBUF:]
        cid = lax.axis_index("c")
        sid = lax.axis_index("s")
        wid = cid * NSUB + sid

        pltpu.sync_copy(z_hbm.at[pl.ds(sid * ROWS_PER_SUB, ROWS_PER_SUB)],
                        acc_sh.at[pl.ds(sid * ROWS_PER_SUB, ROWS_PER_SUB)])
        plsc.subcore_barrier()

        def idx_start(c, b):
            pltpu.async_copy(e_hbm.at[wid].at[c], ib[b], isem[b])

        def idx_wait(c, b):
            pltpu.make_async_copy(e_hbm.at[wid].at[c], ib[b], isem[b]).wait()

        def gather_start(c, b):
            pltpu.async_copy(h_hbm.at[ib[b].at[0]], rows[b], gsem[b])

        def gather_wait(c, b):
            pltpu.make_async_copy(h_hbm.at[ib[b].at[0]], rows[b],
                                  gsem[b]).wait()

        def scatter_start(c, b):
            pltpu.async_copy(rows[b], acc_sh.at[dstb[b].at[0]], ssem[b],
                             add=True)

        def scatter_wait(c, b):
            pltpu.make_async_copy(rows[b], acc_sh.at[dstb[b].at[0]],
                                  ssem[b]).wait()

        def scale(b):
            two = jnp.full((16,), 2, jnp.int32)

            @plsc.parallel_loop(0, K, step=4, unroll=4)
            def _(e0):
                base = jnp.full((16,), e0, jnp.int32)
                for t in range(4):
                    ee = base + t
                    wsplat = plsc.bitcast(
                        plsc.load_gather(ib[b], [two, ee]), jnp.float32)
                    for g in range(d_active // 16):
                        sl = (e0 + t, pl.ds(g * 16, 16))
                        rows[b][sl] = rows[b][sl] * wsplat

        def save_dst(b):
            for g in range(K // 16):
                dstb[b][0, pl.ds(g * 16, 16)] = ib[b][1, pl.ds(g * 16, 16)]

        idx_start(0, 0)
        idx_start(1, 1)
        idx_start(2, 2)
        idx_wait(0, 0)
        gather_start(0, 0)
        idx_wait(1, 1)
        gather_start(1, 1)

        @pl.loop(0, NCH, step=NBUF)
        def _(c0):
            for b in range(NBUF):
                c = c0 + b
                gather_wait(c, b)
                scale(b)
                save_dst(b)
                scatter_start(c, b)

                @pl.when(c + 3 < NCH)
                def _():
                    idx_start(c + 3, b)

                cn = c + 2

                @pl.when(cn < NCH)
                def _():
                    @pl.when(c >= 1)
                    def _():
                        scatter_wait(c - 1, (b - 1) % NBUF)
                    idx_wait(cn, (b + 2) % NBUF)
                    gather_start(cn, (b + 2) % NBUF)

        for i in range(NBUF):
            c = NCH - NBUF + i
            scatter_wait(c, c % NBUF)

        plsc.subcore_barrier()

        pltpu.sync_copy(
            acc_sh.at[pl.ds(sid * ROWS_PER_SUB, ROWS_PER_SUB)],
            out_hbm.at[cid].at[pl.ds(sid * ROWS_PER_SUB, ROWS_PER_SUB)])

    return k(h, edges, zeros)


def kernel(x, edge_index, edge_weight, W1, b1, W2, b2):
    pad = E_PAD - N_EDGES
    pad_dst = N_NODES + (jnp.arange(pad, dtype=jnp.int32) % (N_PAD - N_NODES))
    pad_src = jnp.arange(pad, dtype=jnp.int32) % N_NODES
    dst = jnp.concatenate(
        [edge_index[0].astype(jnp.int32), pad_dst]).reshape(NW, NCH, K)
    src = jnp.concatenate(
        [edge_index[1].astype(jnp.int32), pad_src]).reshape(NW, NCH, K)
    wbits = lax.bitcast_convert_type(
        jnp.pad(edge_weight, (0, pad)), jnp.int32).reshape(NW, NCH, K)
    edges = jnp.stack([src, dst, wbits], axis=2)
    z1 = jnp.zeros((N_PAD, D_HID), jnp.float32)
    W2p = jnp.pad(W2, ((0, D_HID - N_CLASSES), (0, 0)))
    b2p = jnp.pad(b2, (0, D_HID - N_CLASSES)).reshape(1, D_HID)

    h = _linear1(x, W1, b1.reshape(1, D_HID))
    p = _spmm_sc(h, edges, z1, D_HID, D_HID)
    h2 = _relu_linear2(p, W2p, b2p)
    q = _spmm_sc(h2, edges, z1, D_HID, N_CLASSES)
    return _sum_partials(q)


# speedup vs baseline: 1.0643x; 1.0562x over previous
import dataclasses
import functools

import jax
import jax.numpy as jnp
from jax import lax
from jax.experimental import pallas as pl
from jax.experimental.pallas import tpu as pltpu
from jax.experimental.pallas import tpu_sc as plsc

N_NODES = 10000
N_EDGES = 320000
D_IN = 128
D_HID = 128
N_CLASSES = 64

NCORE = 2
NSUB = 16
NW = NCORE * NSUB
K = 96
NBUF = 3
NCH = 105
E_PAD = NW * NCH * K
N_PAD = 10240
ROWS_PER_SUB = N_PAD // NSUB


def _linear1(x, W1, b1):
    def body(x_ref, w_ref, b_ref, o_ref):
        o_ref[...] = lax.dot_general(
            x_ref[...], w_ref[...], (((1,), (1,)), ((), ())),
            preferred_element_type=jnp.float32) + b_ref[...]

    return pl.pallas_call(
        body,
        grid=(10,),
        in_specs=[
            pl.BlockSpec((N_NODES // 10, D_IN), lambda i: (i, 0)),
            pl.BlockSpec((D_HID, D_IN), lambda i: (0, 0)),
            pl.BlockSpec((1, D_HID), lambda i: (0, 0)),
        ],
        out_specs=pl.BlockSpec((N_NODES // 10, D_HID), lambda i: (i, 0)),
        out_shape=jax.ShapeDtypeStruct((N_NODES, D_HID), jnp.float32),
    )(x, W1, b1)


def _relu_linear2(p, W2p, b2p):
    def body(p_ref, w_ref, b_ref, o_ref):
        h = jnp.maximum(p_ref[0] + p_ref[1], 0.0)
        o_ref[...] = lax.dot_general(
            h, w_ref[...], (((1,), (1,)), ((), ())),
            preferred_element_type=jnp.float32) + b_ref[...]

    return pl.pallas_call(
        body,
        grid=(10,),
        in_specs=[
            pl.BlockSpec((NCORE, N_NODES // 10, D_HID), lambda i: (0, i, 0)),
            pl.BlockSpec((D_HID, D_HID), lambda i: (0, 0)),
            pl.BlockSpec((1, D_HID), lambda i: (0, 0)),
        ],
        out_specs=pl.BlockSpec((N_NODES // 10, D_HID), lambda i: (i, 0)),
        out_shape=jax.ShapeDtypeStruct((N_NODES, D_HID), jnp.float32),
    )(p, W2p, b2p)


def _sum_partials(q):
    def body(q_ref, o_ref):
        o_ref[...] = q_ref[0, :, :N_CLASSES] + q_ref[1, :, :N_CLASSES]

    return pl.pallas_call(
        body,
        grid=(10,),
        in_specs=[pl.BlockSpec((NCORE, N_NODES // 10, D_HID),
                               lambda i: (0, i, 0))],
        out_specs=pl.BlockSpec((N_NODES // 10, N_CLASSES), lambda i: (i, 0)),
        out_shape=jax.ShapeDtypeStruct((N_NODES, N_CLASSES), jnp.float32),
    )(q)


def _spmm_sc(h, src3, dst3, w3, zeros, d, d_active):
    mesh = plsc.VectorSubcoreMesh(core_axis_name="c", subcore_axis_name="s")
    cp = pltpu.CompilerParams()
    if "needs_layout_passes" in pltpu.CompilerParams.__dataclass_fields__:
        cp = dataclasses.replace(cp, needs_layout_passes=False)

    @functools.partial(
        pl.kernel,
        out_type=jax.ShapeDtypeStruct((NCORE, N_PAD, d), jnp.float32),
        mesh=mesh,
        compiler_params=cp,
        scratch_types=[
            pltpu.VMEM_SHARED((N_PAD, d), jnp.float32),
        ] + [pltpu.VMEM((K, d), jnp.float32) for _ in range(NBUF)]
          + [pltpu.VMEM((2, K), jnp.int32) for _ in range(NBUF)]
          + [pltpu.VMEM((1, K), jnp.float32) for _ in range(NBUF)]
          + [pltpu.VMEM((1, K), jnp.int32) for _ in range(NBUF)]
          + [pltpu.SemaphoreType.DMA for _ in range(3 * NBUF)],
    )
    def k(h_hbm, s_hbm, d_hbm, w_hbm, z_hbm, out_hbm, acc_sh, *bufs_and_sems):
        rows = bufs_and_sems[:NBUF]
        ib = bufs_and_sems[NBUF:2 * NBUF]
        wb = bufs_and_sems[2 * NBUF:3 * NBUF]
        dstb = bufs_and_sems[3 * NBUF:4 * NBUF]
        sems = bufs_and_sems[4 * NBUF:]
        gsem = sems[:NBUF]
        ssem = sems[NBUF:2 * NBUF]
        isem = sems[2 * NBUF:]
        cid = lax.axis_index("c")
        sid = lax.axis_index("s")
        wid = cid * NSUB + sid

        pltpu.sync_copy(z_hbm.at[pl.ds(sid * ROWS_PER_SUB, ROWS_PER_SUB)],
                        acc_sh.at[pl.ds(sid * ROWS_PER_SUB, ROWS_PER_SUB)])
        plsc.subcore_barrier()

        def idx_start(c, b):
            pltpu.async_copy(s_hbm.at[wid].at[c], ib[b].at[0], isem[b])
            pltpu.async_copy(d_hbm.at[wid].at[c], ib[b].at[1], isem[b])
            pltpu.async_copy(w_hbm.at[wid].at[c], wb[b].at[0], isem[b])

        def idx_wait(c, b):
            pltpu.make_async_copy(s_hbm.at[wid].at[c], ib[b].at[0],
                                  isem[b]).wait()
            pltpu.make_async_copy(d_hbm.at[wid].at[c], ib[b].at[1],
                                  isem[b]).wait()
            pltpu.make_async_copy(w_hbm.at[wid].at[c], wb[b].at[0],
                                  isem[b]).wait()

        def gather_start(c, b):
            pltpu.async_copy(h_hbm.at[ib[b].at[0]], rows[b], gsem[b])

        def gather_wait(c, b):
            pltpu.make_async_copy(h_hbm.at[ib[b].at[0]], rows[b],
                                  gsem[b]).wait()

        def scatter_start(c, b):
            pltpu.async_copy(rows[b], acc_sh.at[dstb[b].at[0]], ssem[b],
                             add=True)

        def scatter_wait(c, b):
            pltpu.make_async_copy(rows[b], acc_sh.at[dstb[b].at[0]],
                                  ssem[b]).wait()

        def scale(b):
            zero = jnp.full((16,), 0, jnp.int32)

            @plsc.parallel_loop(0, K, step=4, unroll=2)
            def _(e0):
                base = jnp.full((16,), e0, jnp.int32)
                for t in range(4):
                    ee = base + t
                    wsplat = plsc.load_gather(wb[b], [zero, ee])
                    for g in range(d_active // 16):
                        sl = (e0 + t, pl.ds(g * 16, 16))
                        rows[b][sl] = rows[b][sl] * wsplat

        def save_dst(b):
            for g in range(K // 16):
                dstb[b][0, pl.ds(g * 16, 16)] = ib[b][1, pl.ds(g * 16, 16)]

        idx_start(0, 0)
        idx_start(1, 1)
        idx_start(2, 2)
        idx_wait(0, 0)
        gather_start(0, 0)
        idx_wait(1, 1)
        gather_start(1, 1)

        @pl.loop(0, NCH, step=NBUF)
        def _(c0):
            for b in range(NBUF):
                c = c0 + b
                gather_wait(c, b)
                scale(b)
                save_dst(b)
                scatter_start(c, b)

                @pl.when(c + 3 < NCH)
                def _():
                    idx_start(c + 3, b)

                cn = c + 2

                @pl.when(cn < NCH)
                def _():
                    @pl.when(c >= 1)
                    def _():
                        scatter_wait(c - 1, (b - 1) % NBUF)
                    idx_wait(cn, (b + 2) % NBUF)
                    gather_start(cn, (b + 2) % NBUF)

        for i in range(NBUF):
            c = NCH - NBUF + i
            scatter_wait(c, c % NBUF)

        plsc.subcore_barrier()

        pltpu.sync_copy(
            acc_sh.at[pl.ds(sid * ROWS_PER_SUB, ROWS_PER_SUB)],
            out_hbm.at[cid].at[pl.ds(sid * ROWS_PER_SUB, ROWS_PER_SUB)])

    return k(h, src3, dst3, w3, zeros)


def kernel(x, edge_index, edge_weight, W1, b1, W2, b2):
    pad = E_PAD - N_EDGES
    pad_dst = N_NODES + (jnp.arange(pad, dtype=jnp.int32) % (N_PAD - N_NODES))
    pad_src = jnp.arange(pad, dtype=jnp.int32) % N_NODES
    dst = jnp.concatenate(
        [edge_index[0].astype(jnp.int32), pad_dst]).reshape(NW, NCH, K)
    src = jnp.concatenate(
        [edge_index[1].astype(jnp.int32), pad_src]).reshape(NW, NCH, K)
    w = jnp.pad(edge_weight, (0, pad)).reshape(NW, NCH, K)
    z1 = jnp.zeros((N_PAD, D_HID), jnp.float32)
    W2p = jnp.pad(W2, ((0, D_HID - N_CLASSES), (0, 0)))
    b2p = jnp.pad(b2, (0, D_HID - N_CLASSES)).reshape(1, D_HID)

    h = _linear1(x, W1, b1.reshape(1, D_HID))
    p = _spmm_sc(h, src, dst, w, z1, D_HID, D_HID)
    h2 = _relu_linear2(p, W2p, b2p)
    q = _spmm_sc(h2, src, dst, w, z1, D_HID, N_CLASSES)
    return _sum_partials(q)


# speedup vs baseline: 1.1306x; 1.0623x over previous
import dataclasses
import functools

import jax
import jax.numpy as jnp
from jax import lax
from jax.experimental import pallas as pl
from jax.experimental.pallas import tpu as pltpu
from jax.experimental.pallas import tpu_sc as plsc

N_NODES = 10000
N_EDGES = 320000
D_IN = 128
D_HID = 128
N_CLASSES = 64

NCORE = 2
NSUB = 16
NW = NCORE * NSUB
K = 128
NBUF = 3
NCH = 79
NCH_MAIN = 78
E_PAD = NW * NCH * K
SLAB = 632
SLAB_LAST = N_NODES - (NSUB - 1) * SLAB


def _linear1(x, W1, b1):
    def body(x_ref, w_ref, b_ref, o_ref):
        o_ref[...] = lax.dot_general(
            x_ref[...], w_ref[...], (((1,), (1,)), ((), ())),
            preferred_element_type=jnp.float32) + b_ref[...]

    return pl.pallas_call(
        body,
        grid=(10,),
        in_specs=[
            pl.BlockSpec((N_NODES // 10, D_IN), lambda i: (i, 0)),
            pl.BlockSpec((D_HID, D_IN), lambda i: (0, 0)),
            pl.BlockSpec((1, D_HID), lambda i: (0, 0)),
        ],
        out_specs=pl.BlockSpec((N_NODES // 10, D_HID), lambda i: (i, 0)),
        out_shape=jax.ShapeDtypeStruct((N_NODES, D_HID), jnp.float32),
    )(x, W1, b1)


def _relu_linear2(p, W2p, b2p):
    def body(p_ref, w_ref, b_ref, o_ref):
        h = jnp.maximum(p_ref[0] + p_ref[1], 0.0)
        o_ref[...] = lax.dot_general(
            h, w_ref[...], (((1,), (1,)), ((), ())),
            preferred_element_type=jnp.float32) + b_ref[...]

    return pl.pallas_call(
        body,
        grid=(10,),
        in_specs=[
            pl.BlockSpec((NCORE, N_NODES // 10, D_HID), lambda i: (0, i, 0)),
            pl.BlockSpec((D_HID, D_HID), lambda i: (0, 0)),
            pl.BlockSpec((1, D_HID), lambda i: (0, 0)),
        ],
        out_specs=pl.BlockSpec((N_NODES // 10, D_HID), lambda i: (i, 0)),
        out_shape=jax.ShapeDtypeStruct((N_NODES, D_HID), jnp.float32),
    )(p, W2p, b2p)


def _sum_partials(q):
    def body(q_ref, o_ref):
        o_ref[...] = q_ref[0, :, :N_CLASSES] + q_ref[1, :, :N_CLASSES]

    return pl.pallas_call(
        body,
        grid=(10,),
        in_specs=[pl.BlockSpec((NCORE, N_NODES // 10, D_HID),
                               lambda i: (0, i, 0))],
        out_specs=pl.BlockSpec((N_NODES // 10, N_CLASSES), lambda i: (i, 0)),
        out_shape=jax.ShapeDtypeStruct((N_NODES, N_CLASSES), jnp.float32),
    )(q)


def _spmm_sc(h, ei, ew, tsrc, tdst, tw, zeros, d, d_active):
    mesh = plsc.VectorSubcoreMesh(core_axis_name="c", subcore_axis_name="s")
    cp = pltpu.CompilerParams()
    if "needs_layout_passes" in pltpu.CompilerParams.__dataclass_fields__:
        cp = dataclasses.replace(cp, needs_layout_passes=False)

    @functools.partial(
        pl.kernel,
        out_type=jax.ShapeDtypeStruct((NCORE, N_NODES, d), jnp.float32),
        mesh=mesh,
        compiler_params=cp,
        scratch_types=[
            pltpu.VMEM_SHARED((N_NODES, d), jnp.float32),
        ] + [pltpu.VMEM((K, d), jnp.float32) for _ in range(NBUF)]
          + [pltpu.VMEM((2, K), jnp.int32) for _ in range(NBUF)]
          + [pltpu.VMEM((1, K), jnp.float32) for _ in range(NBUF)]
          + [pltpu.VMEM((1, K), jnp.int32) for _ in range(NBUF)]
          + [pltpu.SemaphoreType.DMA for _ in range(3 * NBUF)],
    )
    def k(h_hbm, ei_hbm, ew_hbm, ts_hbm, td_hbm, tw_hbm, z_hbm, out_hbm,
          acc_sh, *bufs_and_sems):
        rows = bufs_and_sems[:NBUF]
        ib = bufs_and_sems[NBUF:2 * NBUF]
        wb = bufs_and_sems[2 * NBUF:3 * NBUF]
        dstb = bufs_and_sems[3 * NBUF:4 * NBUF]
        sems = bufs_and_sems[4 * NBUF:]
        gsem = sems[:NBUF]
        ssem = sems[NBUF:2 * NBUF]
        isem = sems[2 * NBUF:]
        cid = lax.axis_index("c")
        sid = lax.axis_index("s")
        wid = cid * NSUB + sid

        @pl.when(sid < NSUB - 1)
        def _():
            pltpu.sync_copy(z_hbm.at[pl.ds(sid * SLAB, SLAB)],
                            acc_sh.at[pl.ds(sid * SLAB, SLAB)])

        @pl.when(sid == NSUB - 1)
        def _():
            pltpu.sync_copy(z_hbm.at[pl.ds((NSUB - 1) * SLAB, SLAB_LAST)],
                            acc_sh.at[pl.ds((NSUB - 1) * SLAB, SLAB_LAST)])

        plsc.subcore_barrier()

        is_tail = wid == NW - 1

        def idx_start(c, b):
            base = (wid * NCH + c) * K

            @pl.when(jnp.logical_not(is_tail))
            def _():
                pltpu.async_copy(ei_hbm.at[1].at[pl.ds(base, K)],
                                 ib[b].at[0], isem[b])
                pltpu.async_copy(ei_hbm.at[0].at[pl.ds(base, K)],
                                 ib[b].at[1], isem[b])
                pltpu.async_copy(ew_hbm.at[pl.ds(base, K)],
                                 wb[b].at[0], isem[b])

            @pl.when(is_tail)
            def _():
                pltpu.async_copy(ts_hbm.at[c], ib[b].at[0], isem[b])
                pltpu.async_copy(td_hbm.at[c], ib[b].at[1], isem[b])
                pltpu.async_copy(tw_hbm.at[c], wb[b].at[0], isem[b])

        def idx_wait(c, b):
            base = (wid * NCH + c) * K

            @pl.when(jnp.logical_not(is_tail))
            def _():
                pltpu.make_async_copy(ei_hbm.at[1].at[pl.ds(base, K)],
                                      ib[b].at[0], isem[b]).wait()
                pltpu.make_async_copy(ei_hbm.at[0].at[pl.ds(base, K)],
                                      ib[b].at[1], isem[b]).wait()
                pltpu.make_async_copy(ew_hbm.at[pl.ds(base, K)],
                                      wb[b].at[0], isem[b]).wait()

            @pl.when(is_tail)
            def _():
                pltpu.make_async_copy(ts_hbm.at[c], ib[b].at[0],
                                      isem[b]).wait()
                pltpu.make_async_copy(td_hbm.at[c], ib[b].at[1],
                                      isem[b]).wait()
                pltpu.make_async_copy(tw_hbm.at[c], wb[b].at[0],
                                      isem[b]).wait()

        def gather_start(c, b):
            pltpu.async_copy(h_hbm.at[ib[b].at[0]], rows[b], gsem[b])

        def gather_wait(c, b):
            pltpu.make_async_copy(h_hbm.at[ib[b].at[0]], rows[b],
                                  gsem[b]).wait()

        def scatter_start(c, b):
            pltpu.async_copy(rows[b], acc_sh.at[dstb[b].at[0]], ssem[b],
                             add=True)

        def scatter_wait(c, b):
            pltpu.make_async_copy(rows[b], acc_sh.at[dstb[b].at[0]],
                                  ssem[b]).wait()

        def scale(b):
            zero = jnp.full((16,), 0, jnp.int32)

            @plsc.parallel_loop(0, K, step=4, unroll=2)
            def _(e0):
                base = jnp.full((16,), e0, jnp.int32)
                for t in range(4):
                    ee = base + t
                    wsplat = plsc.load_gather(wb[b], [zero, ee])
                    for g in range(d_active // 16):
                        sl = (e0 + t, pl.ds(g * 16, 16))
                        rows[b][sl] = rows[b][sl] * wsplat

        def save_dst(b):
            for g in range(K // 16):
                dstb[b][0, pl.ds(g * 16, 16)] = ib[b][1, pl.ds(g * 16, 16)]

        idx_start(0, 0)
        idx_start(1, 1)
        idx_start(2, 2)
        idx_wait(0, 0)
        gather_start(0, 0)
        idx_wait(1, 1)
        gather_start(1, 1)

        @pl.loop(0, NCH_MAIN, step=NBUF)
        def _(c0):
            for b in range(NBUF):
                c = c0 + b
                gather_wait(c, b)
                scale(b)
                save_dst(b)
                scatter_start(c, b)

                @pl.when(c + 3 < NCH)
                def _():
                    idx_start(c + 3, b)

                cn = c + 2

                @pl.when(cn < NCH)
                def _():
                    @pl.when(c >= 1)
                    def _():
                        scatter_wait(c - 1, (b - 1) % NBUF)
                    idx_wait(cn, (b + 2) % NBUF)
                    gather_start(cn, (b + 2) % NBUF)

        bl = NCH_MAIN % NBUF
        gather_wait(NCH - 1, bl)
        scale(bl)
        save_dst(bl)
        scatter_start(NCH - 1, bl)

        for c in range(NCH - NBUF, NCH):
            scatter_wait(c, c % NBUF)

        plsc.subcore_barrier()

        @pl.when(sid < NSUB - 1)
        def _():
            pltpu.sync_copy(acc_sh.at[pl.ds(sid * SLAB, SLAB)],
                            out_hbm.at[cid].at[pl.ds(sid * SLAB, SLAB)])

        @pl.when(sid == NSUB - 1)
        def _():
            pltpu.sync_copy(
                acc_sh.at[pl.ds((NSUB - 1) * SLAB, SLAB_LAST)],
                out_hbm.at[cid].at[pl.ds((NSUB - 1) * SLAB, SLAB_LAST)])

    return k(h, ei, ew, tsrc, tdst, tw, zeros)


def kernel(x, edge_index, edge_weight, W1, b1, W2, b2):
    pad = E_PAD - N_EDGES
    tail0 = (NW - 1) * NCH * K
    ei = edge_index.astype(jnp.int32)
    ew = edge_weight
    pad_dst = jnp.arange(pad, dtype=jnp.int32) % N_NODES
    pad_src = jnp.arange(pad, dtype=jnp.int32) % N_NODES
    tdst = jnp.concatenate([ei[0, tail0:], pad_dst]).reshape(NCH, K)
    tsrc = jnp.concatenate([ei[1, tail0:], pad_src]).reshape(NCH, K)
    tw = jnp.pad(ew[tail0:], (0, pad)).reshape(NCH, K)
    z1 = jnp.zeros((N_NODES, D_HID), jnp.float32)
    W2p = jnp.pad(W2, ((0, D_HID - N_CLASSES), (0, 0)))
    b2p = jnp.pad(b2, (0, D_HID - N_CLASSES)).reshape(1, D_HID)

    h = _linear1(x, W1, b1.reshape(1, D_HID))
    p = _spmm_sc(h, ei, ew, tsrc, tdst, tw, z1, D_HID, D_HID)
    h2 = _relu_linear2(p, W2p, b2p)
    q = _spmm_sc(h2, ei, ew, tsrc, tdst, tw, z1, D_HID, N_CLASSES)
    return _sum_partials(q)


# speedup vs baseline: 1.1569x; 1.0233x over previous
import dataclasses
import functools

import jax
import jax.numpy as jnp
from jax import lax
from jax.experimental import pallas as pl
from jax.experimental.pallas import tpu as pltpu
from jax.experimental.pallas import tpu_sc as plsc

N_NODES = 10000
N_EDGES = 320000
D_IN = 128
D_HID = 128
N_CLASSES = 64

NCORE = 2
NSUB = 16
NW = NCORE * NSUB
K = 128
NBUF = 3
NCH = 79
NCH_MAIN = 78
E_PAD = NW * NCH * K
SLAB = 632
SLAB_LAST = N_NODES - (NSUB - 1) * SLAB


def _linear1(x, W1, b1):
    def body(x_ref, w_ref, b_ref, o_ref):
        o_ref[...] = lax.dot_general(
            x_ref[...], w_ref[...], (((1,), (1,)), ((), ())),
            preferred_element_type=jnp.float32) + b_ref[...]

    return pl.pallas_call(
        body,
        grid=(5,),
        in_specs=[
            pl.BlockSpec((N_NODES // 5, D_IN), lambda i: (i, 0)),
            pl.BlockSpec((D_HID, D_IN), lambda i: (0, 0)),
            pl.BlockSpec((1, D_HID), lambda i: (0, 0)),
        ],
        out_specs=pl.BlockSpec((N_NODES // 5, D_HID), lambda i: (i, 0)),
        out_shape=jax.ShapeDtypeStruct((N_NODES, D_HID), jnp.float32),
    )(x, W1, b1)


def _relu_linear2(p, W2p, b2p):
    def body(p_ref, w_ref, b_ref, o_ref):
        h = jnp.maximum(p_ref[0] + p_ref[1], 0.0)
        o_ref[...] = lax.dot_general(
            h, w_ref[...], (((1,), (1,)), ((), ())),
            preferred_element_type=jnp.float32) + b_ref[...]

    return pl.pallas_call(
        body,
        grid=(5,),
        in_specs=[
            pl.BlockSpec((NCORE, N_NODES // 5, D_HID), lambda i: (0, i, 0)),
            pl.BlockSpec((D_HID, D_HID), lambda i: (0, 0)),
            pl.BlockSpec((1, D_HID), lambda i: (0, 0)),
        ],
        out_specs=pl.BlockSpec((N_NODES // 5, D_HID), lambda i: (i, 0)),
        out_shape=jax.ShapeDtypeStruct((N_NODES, D_HID), jnp.float32),
    )(p, W2p, b2p)


def _sum_partials(q):
    def body(q_ref, o_ref):
        o_ref[...] = q_ref[0, :, :N_CLASSES] + q_ref[1, :, :N_CLASSES]

    return pl.pallas_call(
        body,
        grid=(5,),
        in_specs=[pl.BlockSpec((NCORE, N_NODES // 5, D_HID),
                               lambda i: (0, i, 0))],
        out_specs=pl.BlockSpec((N_NODES // 5, N_CLASSES), lambda i: (i, 0)),
        out_shape=jax.ShapeDtypeStruct((N_NODES, N_CLASSES), jnp.float32),
    )(q)


def _spmm_sc(h, ei, ew, tsrc, tdst, tw, zeros, d, d_active):
    mesh = plsc.VectorSubcoreMesh(core_axis_name="c", subcore_axis_name="s")
    cp = pltpu.CompilerParams()
    if "needs_layout_passes" in pltpu.CompilerParams.__dataclass_fields__:
        cp = dataclasses.replace(cp, needs_layout_passes=False)

    @functools.partial(
        pl.kernel,
        out_type=jax.ShapeDtypeStruct((NCORE, N_NODES, d), jnp.float32),
        mesh=mesh,
        compiler_params=cp,
        scratch_types=[
            pltpu.VMEM_SHARED((N_NODES, d), jnp.float32),
        ] + [pltpu.VMEM((K, d), jnp.float32) for _ in range(NBUF)]
          + [pltpu.VMEM((2, K), jnp.int32) for _ in range(NBUF)]
          + [pltpu.VMEM((1, K), jnp.float32) for _ in range(NBUF)]
          + [pltpu.VMEM((1, K), jnp.int32) for _ in range(NBUF)]
          + [pltpu.SemaphoreType.DMA for _ in range(3 * NBUF)],
    )
    def k(h_hbm, ei_hbm, ew_hbm, ts_hbm, td_hbm, tw_hbm, z_hbm, out_hbm,
          acc_sh, *bufs_and_sems):
        rows = bufs_and_sems[:NBUF]
        ib = bufs_and_sems[NBUF:2 * NBUF]
        wb = bufs_and_sems[2 * NBUF:3 * NBUF]
        dstb = bufs_and_sems[3 * NBUF:4 * NBUF]
        sems = bufs_and_sems[4 * NBUF:]
        gsem = sems[:NBUF]
        ssem = sems[NBUF:2 * NBUF]
        isem = sems[2 * NBUF:]
        cid = lax.axis_index("c")
        sid = lax.axis_index("s")
        wid = cid * NSUB + sid

        @pl.when(sid < NSUB - 1)
        def _():
            pltpu.sync_copy(z_hbm.at[pl.ds(sid * SLAB, SLAB)],
                            acc_sh.at[pl.ds(sid * SLAB, SLAB)])

        @pl.when(sid == NSUB - 1)
        def _():
            pltpu.sync_copy(z_hbm.at[pl.ds((NSUB - 1) * SLAB, SLAB_LAST)],
                            acc_sh.at[pl.ds((NSUB - 1) * SLAB, SLAB_LAST)])

        plsc.subcore_barrier()

        is_tail = wid == NW - 1

        def idx_start(c, b):
            base = (wid * NCH + c) * K

            @pl.when(jnp.logical_not(is_tail))
            def _():
                pltpu.async_copy(ei_hbm.at[1].at[pl.ds(base, K)],
                                 ib[b].at[0], isem[b])
                pltpu.async_copy(ei_hbm.at[0].at[pl.ds(base, K)],
                                 ib[b].at[1], isem[b])
                pltpu.async_copy(ew_hbm.at[pl.ds(base, K)],
                                 wb[b].at[0], isem[b])

            @pl.when(is_tail)
            def _():
                pltpu.async_copy(ts_hbm.at[c], ib[b].at[0], isem[b])
                pltpu.async_copy(td_hbm.at[c], ib[b].at[1], isem[b])
                pltpu.async_copy(tw_hbm.at[c], wb[b].at[0], isem[b])

        def idx_wait(c, b):
            base = (wid * NCH + c) * K

            @pl.when(jnp.logical_not(is_tail))
            def _():
                pltpu.make_async_copy(ei_hbm.at[1].at[pl.ds(base, K)],
                                      ib[b].at[0], isem[b]).wait()
                pltpu.make_async_copy(ei_hbm.at[0].at[pl.ds(base, K)],
                                      ib[b].at[1], isem[b]).wait()
                pltpu.make_async_copy(ew_hbm.at[pl.ds(base, K)],
                                      wb[b].at[0], isem[b]).wait()

            @pl.when(is_tail)
            def _():
                pltpu.make_async_copy(ts_hbm.at[c], ib[b].at[0],
                                      isem[b]).wait()
                pltpu.make_async_copy(td_hbm.at[c], ib[b].at[1],
                                      isem[b]).wait()
                pltpu.make_async_copy(tw_hbm.at[c], wb[b].at[0],
                                      isem[b]).wait()

        def gather_start(c, b):
            pltpu.async_copy(h_hbm.at[ib[b].at[0]], rows[b], gsem[b])

        def gather_wait(c, b):
            pltpu.make_async_copy(h_hbm.at[ib[b].at[0]], rows[b],
                                  gsem[b]).wait()

        def scatter_start(c, b):
            pltpu.async_copy(rows[b], acc_sh.at[dstb[b].at[0]], ssem[b],
                             add=True)

        def scatter_wait(c, b):
            pltpu.make_async_copy(rows[b], acc_sh.at[dstb[b].at[0]],
                                  ssem[b]).wait()

        def scale(b):
            zero = jnp.full((16,), 0, jnp.int32)

            @plsc.parallel_loop(0, K, step=4, unroll=2)
            def _(e0):
                base = jnp.full((16,), e0, jnp.int32)
                for t in range(4):
                    ee = base + t
                    wsplat = plsc.load_gather(wb[b], [zero, ee])
                    for g in range(d_active // 16):
                        sl = (e0 + t, pl.ds(g * 16, 16))
                        rows[b][sl] = rows[b][sl] * wsplat

        def save_dst(b):
            for g in range(K // 16):
                dstb[b][0, pl.ds(g * 16, 16)] = ib[b][1, pl.ds(g * 16, 16)]

        idx_start(0, 0)
        idx_start(1, 1)
        idx_start(2, 2)
        idx_wait(0, 0)
        gather_start(0, 0)
        idx_wait(1, 1)
        gather_start(1, 1)

        @pl.loop(0, NCH_MAIN, step=NBUF)
        def _(c0):
            for b in range(NBUF):
                c = c0 + b
                gather_wait(c, b)
                scale(b)
                save_dst(b)
                scatter_start(c, b)

                @pl.when(c + 3 < NCH)
                def _():
                    idx_start(c + 3, b)

                cn = c + 2

                @pl.when(cn < NCH)
                def _():
                    @pl.when(c >= 1)
                    def _():
                        scatter_wait(c - 1, (b - 1) % NBUF)
                    idx_wait(cn, (b + 2) % NBUF)
                    gather_start(cn, (b + 2) % NBUF)

        bl = NCH_MAIN % NBUF
        gather_wait(NCH - 1, bl)
        scale(bl)
        save_dst(bl)
        scatter_start(NCH - 1, bl)

        for c in range(NCH - NBUF, NCH):
            scatter_wait(c, c % NBUF)

        plsc.subcore_barrier()

        @pl.when(sid < NSUB - 1)
        def _():
            pltpu.sync_copy(acc_sh.at[pl.ds(sid * SLAB, SLAB)],
                            out_hbm.at[cid].at[pl.ds(sid * SLAB, SLAB)])

        @pl.when(sid == NSUB - 1)
        def _():
            pltpu.sync_copy(
                acc_sh.at[pl.ds((NSUB - 1) * SLAB, SLAB_LAST)],
                out_hbm.at[cid].at[pl.ds((NSUB - 1) * SLAB, SLAB_LAST)])

    return k(h, ei, ew, tsrc, tdst, tw, zeros)


def kernel(x, edge_index, edge_weight, W1, b1, W2, b2):
    pad = E_PAD - N_EDGES
    tail0 = (NW - 1) * NCH * K
    ei = edge_index.astype(jnp.int32)
    ew = edge_weight
    pad_dst = jnp.arange(pad, dtype=jnp.int32) % N_NODES
    pad_src = jnp.arange(pad, dtype=jnp.int32) % N_NODES
    tdst = jnp.concatenate([ei[0, tail0:], pad_dst]).reshape(NCH, K)
    tsrc = jnp.concatenate([ei[1, tail0:], pad_src]).reshape(NCH, K)
    tw = jnp.pad(ew[tail0:], (0, pad)).reshape(NCH, K)
    z1 = jnp.zeros((N_NODES, D_HID), jnp.float32)
    W2p = jnp.pad(W2, ((0, D_HID - N_CLASSES), (0, 0)))
    b2p = jnp.pad(b2, (0, D_HID - N_CLASSES)).reshape(1, D_HID)

    h = _linear1(x, W1, b1.reshape(1, D_HID))
    p = _spmm_sc(h, ei, ew, tsrc, tdst, tw, z1, D_HID, D_HID)
    h2 = _relu_linear2(p, W2p, b2p)
    q = _spmm_sc(h2, ei, ew, tsrc, tdst, tw, z1, D_HID, N_CLASSES)
    return _sum_partials(q)


# speedup vs baseline: 1.1797x; 1.0197x over previous
import dataclasses
import functools

import jax
import jax.numpy as jnp
from jax import lax
from jax.experimental import pallas as pl
from jax.experimental.pallas import tpu as pltpu
from jax.experimental.pallas import tpu_sc as plsc

N_NODES = 10000
N_EDGES = 320000
D_IN = 128
D_HID = 128
N_CLASSES = 64

NCORE = 2
NSUB = 16
NW = NCORE * NSUB
K = 128
NBUF = 3
NCH = 79
NCH_MAIN = 78
E_PAD = NW * NCH * K
SLAB = 632
SLAB_LAST = N_NODES - (NSUB - 1) * SLAB


def _linear1(x, W1, b1):
    def body(x_ref, w_ref, b_ref, o_ref):
        o_ref[...] = lax.dot_general(
            x_ref[...], w_ref[...], (((1,), (1,)), ((), ())),
            preferred_element_type=jnp.float32) + b_ref[...]

    return pl.pallas_call(
        body,
        grid=(2,),
        in_specs=[
            pl.BlockSpec((N_NODES // 2, D_IN), lambda i: (i, 0)),
            pl.BlockSpec((D_HID, D_IN), lambda i: (0, 0)),
            pl.BlockSpec((1, D_HID), lambda i: (0, 0)),
        ],
        out_specs=pl.BlockSpec((N_NODES // 2, D_HID), lambda i: (i, 0)),
        out_shape=jax.ShapeDtypeStruct((N_NODES, D_HID), jnp.float32),
    )(x, W1, b1)


def _relu_linear2(p, W2p, b2p):
    def body(p_ref, w_ref, b_ref, o_ref):
        h = jnp.maximum(p_ref[0] + p_ref[1], 0.0)
        o_ref[...] = lax.dot_general(
            h, w_ref[...], (((1,), (1,)), ((), ())),
            preferred_element_type=jnp.float32) + b_ref[...]

    return pl.pallas_call(
        body,
        grid=(2,),
        in_specs=[
            pl.BlockSpec((NCORE, N_NODES // 2, D_HID), lambda i: (0, i, 0)),
            pl.BlockSpec((D_HID, D_HID), lambda i: (0, 0)),
            pl.BlockSpec((1, D_HID), lambda i: (0, 0)),
        ],
        out_specs=pl.BlockSpec((N_NODES // 2, D_HID), lambda i: (i, 0)),
        out_shape=jax.ShapeDtypeStruct((N_NODES, D_HID), jnp.float32),
    )(p, W2p, b2p)


def _sum_partials(q):
    def body(q_ref, o_ref):
        o_ref[...] = q_ref[0, :, :N_CLASSES] + q_ref[1, :, :N_CLASSES]

    return pl.pallas_call(
        body,
        grid=(2,),
        in_specs=[pl.BlockSpec((NCORE, N_NODES // 2, D_HID),
                               lambda i: (0, i, 0))],
        out_specs=pl.BlockSpec((N_NODES // 2, N_CLASSES), lambda i: (i, 0)),
        out_shape=jax.ShapeDtypeStruct((N_NODES, N_CLASSES), jnp.float32),
    )(q)


def _spmm_sc(h, ei, ew, tsrc, tdst, tw, zeros, d, d_active):
    mesh = plsc.VectorSubcoreMesh(core_axis_name="c", subcore_axis_name="s")
    cp = pltpu.CompilerParams()
    if "needs_layout_passes" in pltpu.CompilerParams.__dataclass_fields__:
        cp = dataclasses.replace(cp, needs_layout_passes=False)

    @functools.partial(
        pl.kernel,
        out_type=jax.ShapeDtypeStruct((NCORE, N_NODES, d), jnp.float32),
        mesh=mesh,
        compiler_params=cp,
        scratch_types=[
            pltpu.VMEM_SHARED((N_NODES, d), jnp.float32),
        ] + [pltpu.VMEM((K, d), jnp.float32) for _ in range(NBUF)]
          + [pltpu.VMEM((2, K), jnp.int32) for _ in range(NBUF)]
          + [pltpu.VMEM((1, K), jnp.float32) for _ in range(NBUF)]
          + [pltpu.VMEM((1, K), jnp.int32) for _ in range(NBUF)]
          + [pltpu.SemaphoreType.DMA for _ in range(3 * NBUF)],
    )
    def k(h_hbm, ei_hbm, ew_hbm, ts_hbm, td_hbm, tw_hbm, z_hbm, out_hbm,
          acc_sh, *bufs_and_sems):
        rows = bufs_and_sems[:NBUF]
        ib = bufs_and_sems[NBUF:2 * NBUF]
        wb = bufs_and_sems[2 * NBUF:3 * NBUF]
        dstb = bufs_and_sems[3 * NBUF:4 * NBUF]
        sems = bufs_and_sems[4 * NBUF:]
        gsem = sems[:NBUF]
        ssem = sems[NBUF:2 * NBUF]
        isem = sems[2 * NBUF:]
        cid = lax.axis_index("c")
        sid = lax.axis_index("s")
        wid = cid * NSUB + sid

        @pl.when(sid < NSUB - 1)
        def _():
            pltpu.sync_copy(z_hbm.at[pl.ds(sid * SLAB, SLAB)],
                            acc_sh.at[pl.ds(sid * SLAB, SLAB)])

        @pl.when(sid == NSUB - 1)
        def _():
            pltpu.sync_copy(z_hbm.at[pl.ds((NSUB - 1) * SLAB, SLAB_LAST)],
                            acc_sh.at[pl.ds((NSUB - 1) * SLAB, SLAB_LAST)])

        plsc.subcore_barrier()

        is_tail = wid == NW - 1

        def idx_start(c, b):
            base = (wid * NCH + c) * K

            @pl.when(jnp.logical_not(is_tail))
            def _():
                pltpu.async_copy(ei_hbm.at[1].at[pl.ds(base, K)],
                                 ib[b].at[0], isem[b])
                pltpu.async_copy(ei_hbm.at[0].at[pl.ds(base, K)],
                                 ib[b].at[1], isem[b])
                pltpu.async_copy(ew_hbm.at[pl.ds(base, K)],
                                 wb[b].at[0], isem[b])

            @pl.when(is_tail)
            def _():
                pltpu.async_copy(ts_hbm.at[c], ib[b].at[0], isem[b])
                pltpu.async_copy(td_hbm.at[c], ib[b].at[1], isem[b])
                pltpu.async_copy(tw_hbm.at[c], wb[b].at[0], isem[b])

        def idx_wait(c, b):
            base = (wid * NCH + c) * K

            @pl.when(jnp.logical_not(is_tail))
            def _():
                pltpu.make_async_copy(ei_hbm.at[1].at[pl.ds(base, K)],
                                      ib[b].at[0], isem[b]).wait()
                pltpu.make_async_copy(ei_hbm.at[0].at[pl.ds(base, K)],
                                      ib[b].at[1], isem[b]).wait()
                pltpu.make_async_copy(ew_hbm.at[pl.ds(base, K)],
                                      wb[b].at[0], isem[b]).wait()

            @pl.when(is_tail)
            def _():
                pltpu.make_async_copy(ts_hbm.at[c], ib[b].at[0],
                                      isem[b]).wait()
                pltpu.make_async_copy(td_hbm.at[c], ib[b].at[1],
                                      isem[b]).wait()
                pltpu.make_async_copy(tw_hbm.at[c], wb[b].at[0],
                                      isem[b]).wait()

        def gather_start(c, b):
            pltpu.async_copy(h_hbm.at[ib[b].at[0]], rows[b], gsem[b])

        def gather_wait(c, b):
            pltpu.make_async_copy(h_hbm.at[ib[b].at[0]], rows[b],
                                  gsem[b]).wait()

        def scatter_start(c, b):
            pltpu.async_copy(rows[b], acc_sh.at[dstb[b].at[0]], ssem[b],
                             add=True)

        def scatter_wait(c, b):
            pltpu.make_async_copy(rows[b], acc_sh.at[dstb[b].at[0]],
                                  ssem[b]).wait()

        def scale(b):
            zero = jnp.full((16,), 0, jnp.int32)

            @plsc.parallel_loop(0, K, step=4, unroll=2)
            def _(e0):
                base = jnp.full((16,), e0, jnp.int32)
                for t in range(4):
                    ee = base + t
                    wsplat = plsc.load_gather(wb[b], [zero, ee])
                    for g in range(d_active // 16):
                        sl = (e0 + t, pl.ds(g * 16, 16))
                        rows[b][sl] = rows[b][sl] * wsplat

        def save_dst(b):
            for g in range(K // 16):
                dstb[b][0, pl.ds(g * 16, 16)] = ib[b][1, pl.ds(g * 16, 16)]

        idx_start(0, 0)
        idx_start(1, 1)
        idx_start(2, 2)
        idx_wait(0, 0)
        gather_start(0, 0)
        idx_wait(1, 1)
        gather_start(1, 1)

        @pl.loop(0, NCH_MAIN, step=NBUF)
        def _(c0):
            for b in range(NBUF):
                c = c0 + b
                gather_wait(c, b)
                scale(b)
                save_dst(b)
                scatter_start(c, b)

                @pl.when(c + 3 < NCH)
                def _():
                    idx_start(c + 3, b)

                cn = c + 2

                @pl.when(cn < NCH)
                def _():
                    @pl.when(c >= 1)
                    def _():
                        scatter_wait(c - 1, (b - 1) % NBUF)
                    idx_wait(cn, (b + 2) % NBUF)
                    gather_start(cn, (b + 2) % NBUF)

        bl = NCH_MAIN % NBUF
        gather_wait(NCH - 1, bl)
        scale(bl)
        save_dst(bl)
        scatter_start(NCH - 1, bl)

        for c in range(NCH - NBUF, NCH):
            scatter_wait(c, c % NBUF)

        plsc.subcore_barrier()

        @pl.when(sid < NSUB - 1)
        def _():
            pltpu.sync_copy(acc_sh.at[pl.ds(sid * SLAB, SLAB)],
                            out_hbm.at[cid].at[pl.ds(sid * SLAB, SLAB)])

        @pl.when(sid == NSUB - 1)
        def _():
            pltpu.sync_copy(
                acc_sh.at[pl.ds((NSUB - 1) * SLAB, SLAB_LAST)],
                out_hbm.at[cid].at[pl.ds((NSUB - 1) * SLAB, SLAB_LAST)])

    return k(h, ei, ew, tsrc, tdst, tw, zeros)


def kernel(x, edge_index, edge_weight, W1, b1, W2, b2):
    pad = E_PAD - N_EDGES
    tail0 = (NW - 1) * NCH * K
    ei = edge_index.astype(jnp.int32)
    ew = edge_weight
    pad_dst = jnp.arange(pad, dtype=jnp.int32) % N_NODES
    pad_src = jnp.arange(pad, dtype=jnp.int32) % N_NODES
    tdst = jnp.concatenate([ei[0, tail0:], pad_dst]).reshape(NCH, K)
    tsrc = jnp.concatenate([ei[1, tail0:], pad_src]).reshape(NCH, K)
    tw = jnp.pad(ew[tail0:], (0, pad)).reshape(NCH, K)
    z1 = jnp.zeros((N_NODES, D_HID), jnp.float32)
    W2p = jnp.pad(W2, ((0, D_HID - N_CLASSES), (0, 0)))
    b2p = jnp.pad(b2, (0, D_HID - N_CLASSES)).reshape(1, D_HID)

    h = _linear1(x, W1, b1.reshape(1, D_HID))
    p = _spmm_sc(h, ei, ew, tsrc, tdst, tw, z1, D_HID, D_HID)
    h2 = _relu_linear2(p, W2p, b2p)
    q = _spmm_sc(h2, ei, ew, tsrc, tdst, tw, z1, D_HID, N_CLASSES)
    return _sum_partials(q)
